# baseline (speedup 1.0000x reference)
import functools

import jax
import jax.numpy as jnp
from jax import lax
from jax.experimental import pallas as pl
from jax.experimental.pallas import tpu as pltpu
from jax.experimental.pallas import tpu_sc as plsc

_NW = 32
_BPW = 8
_TS = 8
_BIG = 1.0e9


def _goffs(N):
    offs = list(range(0, N - 15, 16))
    if offs[-1] + 16 < N:
        offs.append(N - 16)
    return offs


def _sc_first_spike(T, B, N):
    nstage = T // _TS
    ngrp = len(_goffs(N))
    tail = N % 16 != 0
    mesh = plsc.VectorSubcoreMesh(core_axis_name="c", subcore_axis_name="s")

    ngrp_main = ngrp - 1 if tail else ngrp
    tail_off = N - 16

    def goff_of(g):
        return pl.multiple_of(g * 16, 16)

    def body(x_hbm, out_hbm, buf, outbuf, flag, sem):
        cid = lax.axis_index("c")
        sid = lax.axis_index("s")
        wid = sid * 2 + cid
        b0 = pl.multiple_of(wid * _BPW, 8)

        for bb in range(_BPW):
            def init_grp(g, c, bb=bb):
                outbuf[bb, pl.ds(goff_of(g), 16)] = jnp.full(
                    (16,), _BIG, jnp.float32
                )
                return c

            lax.fori_loop(0, ngrp_main, init_grp, 0)
            if tail:
                outbuf[bb, pl.ds(tail_off, 16)] = jnp.full(
                    (16,), _BIG, jnp.float32
                )

        flag[0] = jnp.int32(0)

        def stage_step(stage, carry):
            @pl.when(flag[0] == 0)
            def _do_stage():
                t_base = (stage * _TS).astype(jnp.float32)
                pltpu.async_copy(
                    x_hbm.at[pl.ds(stage * _TS, _TS), pl.ds(b0, _BPW), :],
                    buf,
                    sem,
                ).wait()
                sil = jnp.zeros((16,), jnp.int32)
                for bb in range(_BPW):
                    def up_goff(goff, s, bb=bb, t_base=t_base):
                        acc = jnp.full((16,), _BIG, jnp.float32)
                        for t in range(_TS - 1, -1, -1):
                            x = buf[t, bb, pl.ds(goff, 16)]
                            acc = jnp.where(
                                x > 0.5, jnp.float32(t) + t_base, acc
                            )
                        old = outbuf[bb, pl.ds(goff, 16)]
                        new = jnp.where(old < _BIG, old, acc)
                        outbuf[bb, pl.ds(goff, 16)] = new
                        return s | jnp.where(new >= _BIG, 1, 0)

                    sil = lax.fori_loop(
                        0, ngrp_main,
                        lambda g, s, f=up_goff: f(goff_of(g), s), sil
                    )
                    if tail:
                        sil = up_goff(tail_off, sil)
                any_s = sil[0]
                for lane in range(1, 16):
                    any_s = any_s | sil[lane]
                flag[0] = jnp.where(any_s > 0, 0, 1)
            return carry

        lax.fori_loop(0, nstage, stage_step, 0)

        for bb in range(_BPW):
            def fin_goff(goff, bb=bb):
                v = outbuf[bb, pl.ds(goff, 16)]
                outbuf[bb, pl.ds(goff, 16)] = jnp.minimum(
                    v, jnp.float32(T - 1)
                )

            def fin_grp(g, c):
                fin_goff(goff_of(g))
                return c

            lax.fori_loop(0, ngrp_main, fin_grp, 0)
            if tail:
                fin_goff(tail_off)

        pltpu.async_copy(
            outbuf, out_hbm.at[pl.ds(b0, _BPW), :], sem
        ).wait()

    return pl.kernel(
        body,
        mesh=mesh,
        compiler_params=pltpu.CompilerParams(use_tc_tiling_on_sc=True),
        out_type=jax.ShapeDtypeStruct((B, N), jnp.float32),
        scratch_types=[
            pltpu.VMEM((_TS, _BPW, N), jnp.float32),
            pltpu.VMEM((_BPW, N), jnp.float32),
            pltpu.SMEM((1,), jnp.int32),
            pltpu.SemaphoreType.DMA,
        ],
    )


def _tgt_krnl(tgt_ref, out_ref, *, T):
    tg = tgt_ref[...]
    out_ref[...] = jnp.where(tg < 0, tg + T, tg)


def kernel(spk_out, targets):
    T, B, N = spk_out.shape

    first = _sc_first_spike(T, B, N)(spk_out)

    tgt_out = pl.pallas_call(
        functools.partial(_tgt_krnl, T=T),
        out_shape=jax.ShapeDtypeStruct((B, N), jnp.float32),
    )(targets)

    return first, tgt_out


# speedup vs baseline: 2.8323x; 2.8323x over previous
import functools

import jax
import jax.numpy as jnp
from jax import lax
from jax.experimental import pallas as pl
from jax.experimental.pallas import tpu as pltpu
from jax.experimental.pallas import tpu_sc as plsc

_NW = 32
_NB = 8
_TS = 8
_BIG = 1.0e9


def _sc_first_spike(T, N, B):
    nstage = T // _TS
    nblk = N // _NB
    rounds = -(-nblk // _NW)
    ngrp = B // 16
    mesh = plsc.VectorSubcoreMesh(core_axis_name="c", subcore_axis_name="s")

    def body(x_hbm, out_hbm, buf, outbuf, flag, sem):
        cid = lax.axis_index("c")
        sid = lax.axis_index("s")
        wid = sid * 2 + cid

        def process_block(blk):
            n0 = pl.multiple_of(blk * _NB, 8)

            big = jnp.full((16,), _BIG, jnp.float32)
            for nn in range(_NB):
                for g in range(ngrp):
                    outbuf[nn, pl.ds(g * 16, 16)] = big

            flag[0] = jnp.int32(0)

            def stage_step(stage, carry):
                @pl.when(flag[0] == 0)
                def _do_stage():
                    t_base = (stage * _TS).astype(jnp.float32)
                    pltpu.async_copy(
                        x_hbm.at[pl.ds(stage * _TS, _TS),
                                 pl.ds(n0, _NB), :],
                        buf,
                        sem,
                    ).wait()
                    sil = jnp.zeros((16,), jnp.int32)
                    for nn in range(_NB):
                        for g in range(ngrp):
                            goff = g * 16
                            acc = big
                            for t in range(_TS - 1, -1, -1):
                                x = buf[t, nn, pl.ds(goff, 16)]
                                acc = jnp.where(
                                    x > 0.5, jnp.float32(t) + t_base, acc
                                )
                            old = outbuf[nn, pl.ds(goff, 16)]
                            new = jnp.where(old < _BIG, old, acc)
                            outbuf[nn, pl.ds(goff, 16)] = new
                            sil = sil | jnp.where(new >= _BIG, 1, 0)
                    any_s = sil[0]
                    for lane in range(1, 16):
                        any_s = any_s | sil[lane]
                    flag[0] = jnp.where(any_s > 0, 0, 1)
                return carry

            lax.fori_loop(0, nstage, stage_step, 0)

            tmax = jnp.full((16,), jnp.float32(T - 1), jnp.float32)
            for nn in range(_NB):
                for g in range(ngrp):
                    goff = g * 16
                    v = outbuf[nn, pl.ds(goff, 16)]
                    outbuf[nn, pl.ds(goff, 16)] = jnp.minimum(v, tmax)

            pltpu.async_copy(
                outbuf, out_hbm.at[pl.ds(n0, _NB), :], sem
            ).wait()

        def round_body(k, carry):
            blk = k * _NW + wid

            @pl.when(blk < nblk)
            def _do():
                process_block(blk)
            return carry

        lax.fori_loop(0, rounds, round_body, 0)

    return pl.kernel(
        body,
        mesh=mesh,
        out_type=jax.ShapeDtypeStruct((N, B), jnp.float32),
        scratch_types=[
            pltpu.VMEM((_TS, _NB, B), jnp.float32),
            pltpu.VMEM((_NB, B), jnp.float32),
            pltpu.SMEM((1,), jnp.int32),
            pltpu.SemaphoreType.DMA,
        ],
    )


def _tgt_krnl(tgt_ref, out_ref, *, T):
    tg = tgt_ref[...]
    out_ref[...] = jnp.where(tg < 0, tg + T, tg)


def kernel(spk_out, targets):
    T, B, N = spk_out.shape

    spk_t = jnp.transpose(spk_out, (0, 2, 1))
    first_t = _sc_first_spike(T, N, B)(spk_t)

    tgt_out_t = pl.pallas_call(
        functools.partial(_tgt_krnl, T=T),
        out_shape=jax.ShapeDtypeStruct((N, B), jnp.float32),
    )(targets.T)

    return first_t.T, tgt_out_t.T
